# baseline (device time: 185017 ns/iter reference)
import jax
import jax.numpy as jnp
from jax import lax
from jax.experimental import pallas as pl
from jax.experimental.pallas import tpu as pltpu

N_DEV = 4
M_PER = 1024
K = 4096
N = 8192
N_BLK = 512


def _a2a(x_shard):

    def body(x_ref, out_ref, comm_ref, send_sems, recv_sems):
        my = lax.axis_index("i")

        barrier = pltpu.get_barrier_semaphore()
        for off in (1, 2, 3):
            peer = (my + off) % N_DEV
            pl.semaphore_signal(
                barrier, inc=1,
                device_id=(peer,), device_id_type=pl.DeviceIdType.MESH,
            )
        pl.semaphore_wait(barrier, 3)

        sends = []
        for off in (1, 2, 3):
            tgt = (my + off) % N_DEV
            rdma = pltpu.make_async_remote_copy(
                src_ref=x_ref.at[pl.ds(tgt * M_PER, M_PER), :],
                dst_ref=comm_ref.at[off - 1],
                send_sem=send_sems.at[off - 1],
                recv_sem=recv_sems.at[off - 1],
                device_id=(tgt,),
                device_id_type=pl.DeviceIdType.MESH,
            )
            rdma.start()
            sends.append(rdma)

        out_ref[:, pl.ds(my * M_PER, M_PER)] = x_ref[pl.ds(my * M_PER, M_PER), :]

        for off in (1, 2, 3):
            src = (my - off) % N_DEV
            recv = pltpu.make_async_remote_copy(
                src_ref=comm_ref.at[off - 1],
                dst_ref=comm_ref.at[off - 1],
                send_sem=send_sems.at[off - 1],
                recv_sem=recv_sems.at[off - 1],
                device_id=(my,),
                device_id_type=pl.DeviceIdType.MESH,
            )
            recv.wait_recv()
            out_ref[:, pl.ds(src * M_PER, M_PER)] = comm_ref[off - 1]

        for rdma in sends:
            rdma.wait_send()

    return pl.pallas_call(
        body,
        out_shape=jax.ShapeDtypeStruct((M_PER, K), jnp.float32),
        in_specs=[pl.BlockSpec(memory_space=pltpu.VMEM)],
        out_specs=pl.BlockSpec(memory_space=pltpu.VMEM),
        scratch_shapes=[
            pltpu.VMEM((3, M_PER, M_PER), jnp.float32),
            pltpu.SemaphoreType.DMA((3,)),
            pltpu.SemaphoreType.DMA((3,)),
        ],
        compiler_params=pltpu.CompilerParams(collective_id=0),
    )(x_shard)


def _gemm_silu(x_rows, w_mat):

    def body(x_ref, w_ref, out_ref):
        y = jnp.dot(x_ref[...], w_ref[...], preferred_element_type=jnp.float32)
        out_ref[...] = y * jax.nn.sigmoid(y)

    return pl.pallas_call(
        body,
        grid=(N // N_BLK,),
        in_specs=[
            pl.BlockSpec((M_PER, K), lambda n: (0, 0)),
            pl.BlockSpec((K, N_BLK), lambda n: (0, n)),
        ],
        out_specs=pl.BlockSpec((M_PER, N_BLK), lambda n: (0, n)),
        out_shape=jax.ShapeDtypeStruct((M_PER, N), jnp.float32),
        compiler_params=pltpu.CompilerParams(
            dimension_semantics=("arbitrary",),
        ),
    )(x_rows, w_mat)


def kernel(x, w_mat):
    x_rows = _a2a(x)
    return _gemm_silu(x_rows, w_mat)


# device time: 156104 ns/iter; 1.1852x vs baseline; 1.1852x over previous
import functools

import jax
import jax.numpy as jnp
from jax import lax
from jax.experimental import pallas as pl
from jax.experimental.pallas import tpu as pltpu

N_DEV = 4
M_PER = 1024
K = 4096
N = 8192
KSUB = 512
NC = 2048
N_CHUNKS = N // NC

ORDER = [(0, 0), (0, 1), (1, 0), (3, 0), (1, 1), (3, 1), (2, 0), (2, 1)]


def _fused(x_shard, w_mat):
    def body(x_ref, w_ref, out_ref, comm_ref, wbuf, send_sems, recv_sems,
             wsems):
        my = lax.axis_index("i")

        barrier = pltpu.get_barrier_semaphore()
        for off in (1, 2, 3):
            peer = (my + off) % N_DEV
            pl.semaphore_signal(
                barrier, inc=1,
                device_id=(peer,), device_id_type=pl.DeviceIdType.MESH,
            )
        pl.semaphore_wait(barrier, 3)

        sends = []
        for j in (0, 1):
            for off in (1, 3, 2):
                tgt = (my + off) % N_DEV
                rdma = pltpu.make_async_remote_copy(
                    src_ref=x_ref.at[
                        pl.ds(tgt * M_PER, M_PER), pl.ds(j * KSUB, KSUB)
                    ],
                    dst_ref=comm_ref.at[off, j],
                    send_sem=send_sems.at[off, j],
                    recv_sem=recv_sems.at[off, j],
                    device_id=(tgt,),
                    device_id_type=pl.DeviceIdType.MESH,
                )
                rdma.start()
                sends.append(rdma)

        local_copies = []
        for j in (0, 1):
            cp = pltpu.make_async_copy(
                x_ref.at[pl.ds(my * M_PER, M_PER), pl.ds(j * KSUB, KSUB)],
                comm_ref.at[0, j],
                recv_sems.at[0, j],
            )
            cp.start()
            local_copies.append(cp)

        pairs = [(s, j, c) for (s, j) in ORDER for c in range(N_CHUNKS)]

        def wdma(i):
            s, j, c = pairs[i]
            src_k = ((my - s) % N_DEV) * M_PER + j * KSUB
            return pltpu.make_async_copy(
                w_ref.at[pl.ds(src_k, KSUB), pl.ds(c * NC, NC)],
                wbuf.at[i % 2],
                wsems.at[i % 2],
            )

        dmas = {}
        for i in (0, 1):
            dmas[i] = wdma(i)
            dmas[i].start()

        for i, (s, j, c) in enumerate(pairs):
            if c == 0:
                if s == 0:
                    local_copies[j].wait()
                else:
                    recv = pltpu.make_async_remote_copy(
                        src_ref=comm_ref.at[s, j],
                        dst_ref=comm_ref.at[s, j],
                        send_sem=send_sems.at[s, j],
                        recv_sem=recv_sems.at[s, j],
                        device_id=(my,),
                        device_id_type=pl.DeviceIdType.MESH,
                    )
                    recv.wait_recv()
            dmas[i].wait()
            acc = jnp.dot(
                comm_ref[s, j], wbuf[i % 2],
                preferred_element_type=jnp.float32,
            )
            sl = pl.ds(c * NC, NC)
            if (s, j) == ORDER[0]:
                out_ref[:, sl] = acc
            elif (s, j) == ORDER[-1]:
                y = out_ref[:, sl] + acc
                out_ref[:, sl] = y * jax.nn.sigmoid(y)
            else:
                out_ref[:, sl] = out_ref[:, sl] + acc
            nxt = i + 2
            if nxt < len(pairs):
                dmas[nxt] = wdma(nxt)
                dmas[nxt].start()

        for rdma in sends:
            rdma.wait_send()

    return pl.pallas_call(
        body,
        out_shape=jax.ShapeDtypeStruct((M_PER, N), jnp.float32),
        in_specs=[
            pl.BlockSpec(memory_space=pl.ANY),
            pl.BlockSpec(memory_space=pl.ANY),
        ],
        out_specs=pl.BlockSpec(memory_space=pltpu.VMEM),
        scratch_shapes=[
            pltpu.VMEM((N_DEV, 2, M_PER, KSUB), jnp.float32),
            pltpu.VMEM((2, KSUB, NC), jnp.float32),
            pltpu.SemaphoreType.DMA((N_DEV, 2)),
            pltpu.SemaphoreType.DMA((N_DEV, 2)),
            pltpu.SemaphoreType.DMA((2,)),
        ],
        compiler_params=pltpu.CompilerParams(
            collective_id=0,
            vmem_limit_bytes=100 * 1024 * 1024,
        ),
    )(x_shard, w_mat)


def kernel(x, w_mat):
    return _fused(x, w_mat)
